# baseline (speedup 1.0000x reference)
import functools

import jax
import jax.numpy as jnp
import numpy as np
from jax import lax
from jax.experimental import pallas as pl
from jax.experimental.pallas import tpu as pltpu
from jax.experimental.pallas import tpu_sc as plsc

N = 10000
DEG = 32
K = 8
NB = 8
NCHUNK = N // NB
NWORK = 32
CPW = (NCHUNK + NWORK - 1) // NWORK
_LANES = 16

_DUP_PERM = np.array(list(range(8)) * 2, np.int32)
_SCORE_PERM = np.array([(i // 2) % 8 for i in range(32)], np.int32)


def _feat_perm(R):
    p = np.zeros(R, np.int32)
    if R == 64:
        for g in range(2):
            for l in range(16):
                for d in range(2):
                    p[32 * g + 2 * l + d] = (4 * g + (l >> 3) + 2 * d) * 8 + (l & 7)
    else:
        for g in range(R // 32):
            for l in range(16):
                for d in range(2):
                    p[32 * g + 2 * l + d] = (2 * g + d) * 16 + l
    return p


def _fold_weights(W, a_self, a_neigh, heads):
    eye = jnp.eye(heads, dtype=jnp.float32)
    An = jnp.kron(eye, a_neigh[:, None])
    As = jnp.kron(eye, a_self[:, None])
    R = W.shape[1]
    Sn = (W @ An)[:, _SCORE_PERM]
    Sd = (W @ As)[:, _DUP_PERM]
    return jnp.concatenate([W[:, _feat_perm(R)], Sn, Sd], axis=1)


def _proj_body(R, x_ref, w_ref, t_ref, sdup_ref):
    out = jnp.dot(x_ref[...], w_ref[...], preferred_element_type=jnp.float32)
    t_ref[...] = out[:, :R + 32].astype(jnp.bfloat16)
    sdup_ref[...] = out[:, R + 32:]


def _project(x, Wfull, bn=2000):
    n, d = x.shape
    R = Wfull.shape[1] - 48
    grid = (n + bn - 1) // bn
    return pl.pallas_call(
        functools.partial(_proj_body, R),
        grid=(grid,),
        in_specs=[
            pl.BlockSpec((bn, d), lambda i: (i, 0)),
            pl.BlockSpec((d, R + 48), lambda i: (0, 0)),
        ],
        out_specs=[
            pl.BlockSpec((bn, R + 32), lambda i: (i, 0)),
            pl.BlockSpec((bn, 16), lambda i: (i, 0)),
        ],
        out_shape=[
            jax.ShapeDtypeStruct((n, R + 32), jnp.bfloat16),
            jax.ShapeDtypeStruct((n, 16), jnp.float32),
        ],
    )(x, Wfull)


def _bcast_lane(vec, idxv):
    dnums = lax.GatherDimensionNumbers(
        offset_dims=(), collapsed_slice_dims=(0,), start_index_map=(0,))
    return lax.gather(vec, idxv[:, None], dnums, slice_sizes=(1,),
                      mode=lax.GatherScatterMode.PROMISE_IN_BOUNDS)


def _leaky(e):
    return jnp.maximum(e, 0.01 * e)


def _make_sc_attention(R, final_layer):
    out_dim = 16 if final_layer else R
    tcols = R + 32
    nacc = R // _LANES
    mesh = plsc.VectorSubcoreMesh(core_axis_name="c", subcore_axis_name="s")

    @functools.partial(
        pl.kernel,
        mesh=mesh,
        compiler_params=pltpu.CompilerParams(
            use_tc_tiling_on_sc=False, needs_layout_passes=False),
        out_type=jax.ShapeDtypeStruct((N, out_dim), jnp.float32),
        scratch_types=[
            pltpu.VMEM((CPW, 2, 128), jnp.int32),
            pltpu.VMEM((CPW * NB, 16), jnp.float32),
            pltpu.VMEM((CPW * NB, out_dim), jnp.float32),
            pltpu.VMEM((128, tcols), jnp.bfloat16),
            pltpu.VMEM((128, tcols), jnp.bfloat16),
            pltpu.VMEM((128, tcols), jnp.bfloat16),
            pltpu.VMEM((128, tcols), jnp.bfloat16),
            pltpu.VMEM((DEG, 16), jnp.float32),
            pltpu.SemaphoreType.DMA,
            pltpu.SemaphoreType.DMA,
            pltpu.SemaphoreType.DMA,
            pltpu.SemaphoreType.DMA,
        ],
    )
    def sc_attn(t_hbm, sdup_hbm, nbr_hbm, out_hbm,
                idx_all, sdup_v, out_v, ra0, ra1, rb0, rb1, exm,
                sa0, sa1, sb0, sb1):
        wid = lax.axis_index("s") * 2 + lax.axis_index("c")
        lane = lax.iota(jnp.int32, 16)
        start = jnp.minimum(wid * CPW, NCHUNK - CPW)

        pltpu.sync_copy(nbr_hbm.at[pl.ds(start, CPW)], idx_all)
        pltpu.sync_copy(sdup_hbm.at[pl.ds(start * NB, CPW * NB)], sdup_v)

        def fire(c_local, r0, r1, s0, s1):
            cp0 = pltpu.async_copy(t_hbm.at[idx_all.at[c_local, 0]], r0, s0)
            cp1 = pltpu.async_copy(t_hbm.at[idx_all.at[c_local, 1]], r1, s1)
            return cp0, cp1

        if final_layer:
            bidx = [jnp.full((16,), h, jnp.int32) for h in range(K)]
        else:
            bidx = [2 * j + (lane >> 3) for j in range(nacc)]

        zero = jnp.zeros((16,), jnp.float32)

        def compute(c_local, r0, r1):
            for half, rows in ((0, r0), (1, r1)):

                def node_body(i, _):
                    nrow = c_local * NB + half * (NB // 2) + i
                    sself = sdup_v[nrow, :]
                    base = i * DEG
                    ssum = zero
                    acc = [zero] * nacc
                    for d in range(DEG):
                        j = base + d
                        srow, _ = plsc.unpack(
                            rows[j, pl.ds(R, 32)],
                            format=plsc.PackFormat.INTERLEAVED)
                        ex = jnp.exp(_leaky(sself + srow))
                        ssum = ssum + ex
                        exm[d, :] = ex
                        didx = jnp.full((16,), d, jnp.int32)
                        for g in range(nacc // 2):
                            fa, fb = plsc.unpack(
                                rows[j, pl.ds(32 * g, 32)],
                                format=plsc.PackFormat.INTERLEAVED)
                            wa = plsc.load_gather(exm, [didx, bidx[2 * g]])
                            wb = _bcast_lane(ex, bidx[2 * g + 1])
                            acc[2 * g] = acc[2 * g] + wa * fa
                            acc[2 * g + 1] = acc[2 * g + 1] + wb * fb
                    rs = 1.0 / ssum
                    if final_layer:
                        msum = zero
                        for h in range(K):
                            msum = msum + _bcast_lane(rs, bidx[h]) * acc[h]
                        msum = msum * (1.0 / K)
                        ex = jnp.exp(msum)
                        cs = plsc.cumsum(ex)
                        totv = _bcast_lane(cs, jnp.full((16,), 15, jnp.int32))
                        out_v[nrow, :] = ex / totv
                    else:
                        for r in range(nacc):
                            o = _bcast_lane(rs, bidx[r]) * acc[r]
                            o = jnp.where(o > 0, o,
                                          jnp.exp(jnp.minimum(o, 0.0)) - 1.0)
                            out_v[nrow, pl.ds(r * 16, 16)] = o
                    return 0

                lax.fori_loop(0, NB // 2, node_body, 0)

        fire(0, ra0, ra1, sa0, sa1)

        def kbody2(kk, carry):
            k = 2 * kk
            cb0, cb1 = fire(k + 1, rb0, rb1, sb0, sb1)
            pltpu.make_async_copy(t_hbm.at[idx_all.at[k, 0]], ra0, sa0).wait()
            pltpu.make_async_copy(t_hbm.at[idx_all.at[k, 1]], ra1, sa1).wait()
            compute(k, ra0, ra1)
            knext = jnp.minimum(k + 2, CPW - 1)
            fire(knext, ra0, ra1, sa0, sa1)
            cb0.wait()
            cb1.wait()
            compute(k + 1, rb0, rb1)
            return carry

        lax.fori_loop(0, CPW // 2, kbody2, 0)
        pltpu.make_async_copy(t_hbm.at[idx_all.at[0, 0]], ra0, sa0).wait()
        pltpu.make_async_copy(t_hbm.at[idx_all.at[0, 1]], ra1, sa1).wait()

        pltpu.sync_copy(out_v, out_hbm.at[pl.ds(start * NB, CPW * NB)])

    return sc_attn


_sc_attn1 = _make_sc_attention(64, final_layer=False)
_sc_attn2 = _make_sc_attention(128, final_layer=True)


def kernel(node_features, neighbors, W1, a1_self, a1_neigh, W2, a2_self, a2_neigh):
    nbr3 = neighbors.astype(jnp.int32).reshape(NCHUNK, 2, 128)
    Wf1 = _fold_weights(W1, a1_self, a1_neigh, K)
    Wf2 = _fold_weights(W2, a2_self, a2_neigh, K)

    t1, sdup1 = _project(node_features, Wf1)
    x1 = _sc_attn1(t1, sdup1, nbr3)
    t2, sdup2 = _project(x1, Wf2)
    return _sc_attn2(t2, sdup2, nbr3)


# speedup vs baseline: 2.4723x; 2.4723x over previous
import functools

import jax
import jax.numpy as jnp
import numpy as np
from jax import lax
from jax.experimental import pallas as pl
from jax.experimental.pallas import tpu as pltpu
from jax.experimental.pallas import tpu_sc as plsc

N = 10000
DEG = 32
K = 8
NB = 8
NCHUNK = N // NB
NWORK = 32
CPW = (NCHUNK + NWORK - 1) // NWORK
_LANES = 16

_DUP_PERM = np.array(list(range(8)) * 2, np.int32)
_SCORE_PERM = np.array([(i // 2) % 8 for i in range(32)], np.int32)


def _feat_perm(R):
    p = np.zeros(R, np.int32)
    if R == 64:
        for g in range(2):
            for l in range(16):
                for d in range(2):
                    p[32 * g + 2 * l + d] = (4 * g + (l >> 3) + 2 * d) * 8 + (l & 7)
    else:
        for g in range(R // 32):
            for l in range(16):
                for d in range(2):
                    p[32 * g + 2 * l + d] = (2 * g + d) * 16 + l
    return p


def _fold_weights(W, a_self, a_neigh, heads):
    eye = jnp.eye(heads, dtype=jnp.float32)
    An = jnp.kron(eye, a_neigh[:, None])
    As = jnp.kron(eye, a_self[:, None])
    R = W.shape[1]
    Sn = (W @ An)[:, _SCORE_PERM]
    Sd = (W @ As)[:, _DUP_PERM]
    return jnp.concatenate([W[:, _feat_perm(R)], Sn, Sd], axis=1)


def _proj_body(R, x_ref, w_ref, t_ref, sdup_ref):
    out = jnp.dot(x_ref[...], w_ref[...], preferred_element_type=jnp.float32)
    t_ref[...] = out[:, :R + 32].astype(jnp.bfloat16)
    sdup_ref[...] = out[:, R + 32:]


def _project(x, Wfull, bn=2000):
    n, d = x.shape
    R = Wfull.shape[1] - 48
    grid = (n + bn - 1) // bn
    return pl.pallas_call(
        functools.partial(_proj_body, R),
        grid=(grid,),
        in_specs=[
            pl.BlockSpec((bn, d), lambda i: (i, 0)),
            pl.BlockSpec((d, R + 48), lambda i: (0, 0)),
        ],
        out_specs=[
            pl.BlockSpec((bn, R + 32), lambda i: (i, 0)),
            pl.BlockSpec((bn, 16), lambda i: (i, 0)),
        ],
        out_shape=[
            jax.ShapeDtypeStruct((n, R + 32), jnp.bfloat16),
            jax.ShapeDtypeStruct((n, 16), jnp.float32),
        ],
    )(x, Wfull)


def _bcast_lane(vec, idxv):
    dnums = lax.GatherDimensionNumbers(
        offset_dims=(), collapsed_slice_dims=(0,), start_index_map=(0,))
    return lax.gather(vec, idxv[:, None], dnums, slice_sizes=(1,),
                      mode=lax.GatherScatterMode.PROMISE_IN_BOUNDS)


def _leaky(e):
    return jnp.maximum(e, 0.01 * e)


def _make_sc_attention(R, final_layer):
    out_dim = 16 if final_layer else R
    tcols = R + 32
    nacc = R // _LANES
    mesh = plsc.VectorSubcoreMesh(core_axis_name="c", subcore_axis_name="s")

    @functools.partial(
        pl.kernel,
        mesh=mesh,
        compiler_params=pltpu.CompilerParams(
            use_tc_tiling_on_sc=False, needs_layout_passes=False),
        out_type=jax.ShapeDtypeStruct((N, out_dim), jnp.float32),
        scratch_types=[
            pltpu.VMEM((CPW, 2, 128), jnp.int32),
            pltpu.VMEM((CPW * NB, 16), jnp.float32),
            pltpu.VMEM((CPW * NB, out_dim), jnp.float32),
            pltpu.VMEM((128, tcols), jnp.bfloat16),
            pltpu.VMEM((128, tcols), jnp.bfloat16),
            pltpu.VMEM((128, tcols), jnp.bfloat16),
            pltpu.VMEM((128, tcols), jnp.bfloat16),
            pltpu.SemaphoreType.DMA,
            pltpu.SemaphoreType.DMA,
            pltpu.SemaphoreType.DMA,
            pltpu.SemaphoreType.DMA,
        ],
    )
    def sc_attn(t_hbm, sdup_hbm, nbr_hbm, out_hbm,
                idx_all, sdup_v, out_v, ra0, ra1, rb0, rb1,
                sa0, sa1, sb0, sb1):
        wid = lax.axis_index("s") * 2 + lax.axis_index("c")
        lane = lax.iota(jnp.int32, 16)
        start = jnp.minimum(wid * CPW, NCHUNK - CPW)

        pltpu.sync_copy(nbr_hbm.at[pl.ds(start, CPW)], idx_all)
        pltpu.sync_copy(sdup_hbm.at[pl.ds(start * NB, CPW * NB)], sdup_v)

        def fire(c_local, r0, r1, s0, s1):
            cp0 = pltpu.async_copy(t_hbm.at[idx_all.at[c_local, 0]], r0, s0)
            cp1 = pltpu.async_copy(t_hbm.at[idx_all.at[c_local, 1]], r1, s1)
            return cp0, cp1

        if final_layer:
            bidx = [jnp.full((16,), h, jnp.int32) for h in range(K)]
        else:
            bidx = [2 * j + (lane >> 3) for j in range(nacc)]

        zero = jnp.zeros((16,), jnp.float32)

        def compute(c_local, r0, r1):
            for half, rows in ((0, r0), (1, r1)):

                def node_body(i, _):
                    nrow = c_local * NB + half * (NB // 2) + i
                    sself = sdup_v[nrow, :]
                    base = i * DEG
                    ssum = zero
                    acc = [zero] * nacc
                    for d in range(DEG):
                        j = base + d
                        srow, _ = plsc.unpack(
                            rows[j, pl.ds(R, 32)],
                            format=plsc.PackFormat.INTERLEAVED)
                        ex = jnp.exp(_leaky(sself + srow))
                        ssum = ssum + ex
                        for g in range(nacc // 2):
                            fa, fb = plsc.unpack(
                                rows[j, pl.ds(32 * g, 32)],
                                format=plsc.PackFormat.INTERLEAVED)
                            wa = _bcast_lane(ex, bidx[2 * g])
                            wb = _bcast_lane(ex, bidx[2 * g + 1])
                            acc[2 * g] = acc[2 * g] + wa * fa
                            acc[2 * g + 1] = acc[2 * g + 1] + wb * fb
                    rs = 1.0 / ssum
                    if final_layer:
                        msum = zero
                        for h in range(K):
                            msum = msum + _bcast_lane(rs, bidx[h]) * acc[h]
                        msum = msum * (1.0 / K)
                        ex = jnp.exp(msum)
                        cs = plsc.cumsum(ex)
                        totv = _bcast_lane(cs, jnp.full((16,), 15, jnp.int32))
                        out_v[nrow, :] = ex / totv
                    else:
                        for r in range(nacc):
                            o = _bcast_lane(rs, bidx[r]) * acc[r]
                            o = jnp.where(o > 0, o,
                                          jnp.exp(jnp.minimum(o, 0.0)) - 1.0)
                            out_v[nrow, pl.ds(r * 16, 16)] = o
                    return 0

                lax.fori_loop(0, NB // 2, node_body, 0)

        fire(0, ra0, ra1, sa0, sa1)

        def kbody2(kk, carry):
            k = 2 * kk
            cb0, cb1 = fire(k + 1, rb0, rb1, sb0, sb1)
            pltpu.make_async_copy(t_hbm.at[idx_all.at[k, 0]], ra0, sa0).wait()
            pltpu.make_async_copy(t_hbm.at[idx_all.at[k, 1]], ra1, sa1).wait()
            compute(k, ra0, ra1)
            knext = jnp.minimum(k + 2, CPW - 1)
            fire(knext, ra0, ra1, sa0, sa1)
            cb0.wait()
            cb1.wait()
            compute(k + 1, rb0, rb1)
            return carry

        lax.fori_loop(0, CPW // 2, kbody2, 0)
        pltpu.make_async_copy(t_hbm.at[idx_all.at[0, 0]], ra0, sa0).wait()
        pltpu.make_async_copy(t_hbm.at[idx_all.at[0, 1]], ra1, sa1).wait()

        pltpu.sync_copy(out_v, out_hbm.at[pl.ds(start * NB, CPW * NB)])

    return sc_attn


_sc_attn1 = _make_sc_attention(64, final_layer=False)
_sc_attn2 = _make_sc_attention(128, final_layer=True)


def kernel(node_features, neighbors, W1, a1_self, a1_neigh, W2, a2_self, a2_neigh):
    nbr3 = neighbors.astype(jnp.int32).reshape(NCHUNK, 2, 128)
    Wf1 = _fold_weights(W1, a1_self, a1_neigh, K)
    Wf2 = _fold_weights(W2, a2_self, a2_neigh, K)

    t1, sdup1 = _project(node_features, Wf1)
    x1 = _sc_attn1(t1, sdup1, nbr3)
    t2, sdup2 = _project(x1, Wf2)
    return _sc_attn2(t2, sdup2, nbr3)


# speedup vs baseline: 2.8024x; 1.1335x over previous
import functools

import jax
import jax.numpy as jnp
import numpy as np
from jax import lax
from jax.experimental import pallas as pl
from jax.experimental.pallas import tpu as pltpu
from jax.experimental.pallas import tpu_sc as plsc

N = 10000
DEG = 32
K = 8
NB = 8
NCHUNK = N // NB
NWORK = 32
CPW = (NCHUNK + NWORK - 1) // NWORK
_LANES = 16

_DUP_PERM = np.array(list(range(8)) * 2, np.int32)


def _fold_weights(W, a_self, a_neigh, heads):
    eye = jnp.eye(heads, dtype=jnp.float32)
    An = jnp.kron(eye, a_neigh[:, None])
    As = jnp.kron(eye, a_self[:, None])
    Sn = (W @ An)[:, _DUP_PERM]
    Sd = (W @ As)[:, _DUP_PERM]
    return jnp.concatenate([W, Sn, Sd], axis=1)


def _proj_body(tcols, x_ref, w_ref, t_ref, sdup_ref):
    out = jnp.dot(x_ref[...], w_ref[...], preferred_element_type=jnp.float32)
    t_ref[...] = out[:, :tcols]
    sdup_ref[...] = out[:, tcols:]


def _project(x, Wfull, bn=2000):
    n, d = x.shape
    cols = Wfull.shape[1]
    tcols = cols - 16
    grid = (n + bn - 1) // bn
    return pl.pallas_call(
        functools.partial(_proj_body, tcols),
        grid=(grid,),
        in_specs=[
            pl.BlockSpec((bn, d), lambda i: (i, 0)),
            pl.BlockSpec((d, cols), lambda i: (0, 0)),
        ],
        out_specs=[
            pl.BlockSpec((bn, tcols), lambda i: (i, 0)),
            pl.BlockSpec((bn, 16), lambda i: (i, 0)),
        ],
        out_shape=[
            jax.ShapeDtypeStruct((n, tcols), jnp.float32),
            jax.ShapeDtypeStruct((n, 16), jnp.float32),
        ],
    )(x, Wfull)


def _bcast_lane(vec, idxv):
    dnums = lax.GatherDimensionNumbers(
        offset_dims=(), collapsed_slice_dims=(0,), start_index_map=(0,))
    return lax.gather(vec, idxv[:, None], dnums, slice_sizes=(1,),
                      mode=lax.GatherScatterMode.PROMISE_IN_BOUNDS)


def _leaky(e):
    return jnp.maximum(e, 0.01 * e)


def _make_sc_attention(R, final_layer):
    out_dim = 16 if final_layer else R
    tcols = R + 16
    nacc = R // _LANES
    mesh = plsc.VectorSubcoreMesh(core_axis_name="c", subcore_axis_name="s")

    @functools.partial(
        pl.kernel,
        mesh=mesh,
        compiler_params=pltpu.CompilerParams(
            use_tc_tiling_on_sc=False, needs_layout_passes=False),
        out_type=jax.ShapeDtypeStruct((N, out_dim), jnp.float32),
        scratch_types=[
            pltpu.VMEM((CPW, 2, 128), jnp.int32),
            pltpu.VMEM((CPW * NB, 16), jnp.float32),
            pltpu.VMEM((CPW * NB, out_dim), jnp.float32),
            pltpu.VMEM((128, tcols), jnp.float32),
            pltpu.VMEM((128, tcols), jnp.float32),
            pltpu.VMEM((128, tcols), jnp.float32),
            pltpu.VMEM((128, tcols), jnp.float32),
            pltpu.SemaphoreType.DMA,
            pltpu.SemaphoreType.DMA,
            pltpu.SemaphoreType.DMA,
            pltpu.SemaphoreType.DMA,
        ],
    )
    def sc_attn(t_hbm, sdup_hbm, nbr_hbm, out_hbm,
                idx_all, sdup_v, out_v, ra0, ra1, rb0, rb1,
                sa0, sa1, sb0, sb1):
        wid = lax.axis_index("s") * 2 + lax.axis_index("c")
        lane = lax.iota(jnp.int32, 16)
        start = jnp.minimum(wid * CPW, NCHUNK - CPW)

        pltpu.sync_copy(nbr_hbm.at[pl.ds(start, CPW)], idx_all)
        pltpu.sync_copy(sdup_hbm.at[pl.ds(start * NB, CPW * NB)], sdup_v)

        def fire(c_local, r0, r1, s0, s1):
            cp0 = pltpu.async_copy(t_hbm.at[idx_all.at[c_local, 0]], r0, s0)
            cp1 = pltpu.async_copy(t_hbm.at[idx_all.at[c_local, 1]], r1, s1)
            return cp0, cp1

        if final_layer:
            bidx = [jnp.full((16,), h, jnp.int32) for h in range(K)]
        else:
            bidx = [2 * j + (lane >> 3) for j in range(nacc)]

        zero = jnp.zeros((16,), jnp.float32)

        def compute(c_local, r0, r1):
            for half, rows in ((0, r0), (1, r1)):

                def node_body(i, _):
                    nrow = c_local * NB + half * (NB // 2) + i
                    sself = sdup_v[nrow, :]
                    base = i * DEG
                    ssum = zero
                    acc = [zero] * nacc
                    for d in range(DEG):
                        j = base + d
                        srow = rows[j, pl.ds(R, 16)]
                        ex = jnp.exp(_leaky(sself + srow))
                        ssum = ssum + ex
                        for r in range(nacc):
                            w = _bcast_lane(ex, bidx[r])
                            acc[r] = acc[r] + w * rows[j, pl.ds(r * 16, 16)]
                    rs = 1.0 / ssum
                    if final_layer:
                        msum = zero
                        for h in range(K):
                            msum = msum + _bcast_lane(rs, bidx[h]) * acc[h]
                        msum = msum * (1.0 / K)
                        ex = jnp.exp(msum)
                        cs = plsc.cumsum(ex)
                        totv = _bcast_lane(cs, jnp.full((16,), 15, jnp.int32))
                        out_v[nrow, :] = ex / totv
                    else:
                        for r in range(nacc):
                            o = _bcast_lane(rs, bidx[r]) * acc[r]
                            o = jnp.where(o > 0, o,
                                          jnp.exp(jnp.minimum(o, 0.0)) - 1.0)
                            out_v[nrow, pl.ds(r * 16, 16)] = o
                    return 0

                lax.fori_loop(0, NB // 2, node_body, 0)

        fire(0, ra0, ra1, sa0, sa1)

        def kbody2(kk, carry):
            k = 2 * kk
            cb0, cb1 = fire(k + 1, rb0, rb1, sb0, sb1)
            pltpu.make_async_copy(t_hbm.at[idx_all.at[k, 0]], ra0, sa0).wait()
            pltpu.make_async_copy(t_hbm.at[idx_all.at[k, 1]], ra1, sa1).wait()
            compute(k, ra0, ra1)
            knext = jnp.minimum(k + 2, CPW - 1)
            fire(knext, ra0, ra1, sa0, sa1)
            cb0.wait()
            cb1.wait()
            compute(k + 1, rb0, rb1)
            return carry

        lax.fori_loop(0, CPW // 2, kbody2, 0)
        pltpu.make_async_copy(t_hbm.at[idx_all.at[0, 0]], ra0, sa0).wait()
        pltpu.make_async_copy(t_hbm.at[idx_all.at[0, 1]], ra1, sa1).wait()

        pltpu.sync_copy(out_v, out_hbm.at[pl.ds(start * NB, CPW * NB)])

    return sc_attn


_sc_attn1 = _make_sc_attention(64, final_layer=False)
_sc_attn2 = _make_sc_attention(128, final_layer=True)


def kernel(node_features, neighbors, W1, a1_self, a1_neigh, W2, a2_self, a2_neigh):
    nbr3 = neighbors.astype(jnp.int32).reshape(NCHUNK, 2, 128)
    Wf1 = _fold_weights(W1, a1_self, a1_neigh, K)
    Wf2 = _fold_weights(W2, a2_self, a2_neigh, K)

    t1, sdup1 = _project(node_features, Wf1)
    x1 = _sc_attn1(t1, sdup1, nbr3)
    t2, sdup2 = _project(x1, Wf2)
    return _sc_attn2(t2, sdup2, nbr3)
